# Initial kernel scaffold; baseline (speedup 1.0000x reference)
import jax
import jax.numpy as jnp
from jax.experimental import pallas as pl


def kernel(beta, pred, particle_id, track_params, reconstructable):
    raise NotImplementedError("write your pallas kernel here")


# speedup vs baseline: 2.7834x; 2.7834x over previous
import functools

import jax
import jax.numpy as jnp
from jax import lax
from jax.experimental import pallas as pl
from jax.experimental.pallas import tpu as pltpu
from jax.experimental.pallas import tpu_sc as plsc

_NUM_BINS = 1024
_NSUB = 16
_BPW = _NUM_BINS // _NSUB
_LANES = 128


def _tc_body(beta_ref, p_ref, t_ref, pid_ref, rec_ref, w_ref, wm_ref, c_ref):
    beta = beta_ref[...]
    d = p_ref[...] - t_ref[...]
    mse = jnp.sum(d * d, axis=0)
    m = (pid_ref[...] > 0) & (rec_ref[...] > 0)
    ath = 0.5 * jnp.log((1.0 + beta) / (1.0 - beta))
    w = jnp.where(m, ath * ath, 0.0)
    w_ref[...] = w
    wm_ref[...] = w * mse
    c_ref[...] = m.astype(jnp.float32)


def _sc_body(rows, w_hbm, wm_hbm, c_hbm, pid_hbm,
             aw_hbm, am_hbm, ac_hbm,
             idx_v, wv, mv, cv, zb, ow, om, oc,
             aw, am, ac):
    s = lax.axis_index("s")
    chunk = rows * _LANES
    zero16 = jnp.zeros((16,), jnp.float32)

    for k in range(_BPW // 16):
        zb[pl.ds(16 * k, 16)] = zero16
    pltpu.sync_copy(zb, aw.at[pl.ds(s * _BPW, _BPW)])
    pltpu.sync_copy(zb, am.at[pl.ds(s * _BPW, _BPW)])
    pltpu.sync_copy(zb, ac.at[pl.ds(s * _BPW, _BPW)])

    pltpu.sync_copy(pid_hbm.at[pl.ds(s * chunk, chunk)], idx_v)
    pltpu.sync_copy(w_hbm.at[pl.ds(s * chunk, chunk)], wv)
    pltpu.sync_copy(wm_hbm.at[pl.ds(s * chunk, chunk)], mv)
    pltpu.sync_copy(c_hbm.at[pl.ds(s * chunk, chunk)], cv)
    plsc.subcore_barrier()

    for j in range(rows):
        ji = idx_v.at[pl.ds(j * _LANES, _LANES)]
        pltpu.sync_copy(wv.at[pl.ds(j * _LANES, _LANES)], aw.at[ji], add=True)
        pltpu.sync_copy(mv.at[pl.ds(j * _LANES, _LANES)], am.at[ji], add=True)
        pltpu.sync_copy(cv.at[pl.ds(j * _LANES, _LANES)], ac.at[ji], add=True)
    plsc.subcore_barrier()

    pltpu.sync_copy(aw.at[pl.ds(s * _BPW, _BPW)], ow)
    pltpu.sync_copy(am.at[pl.ds(s * _BPW, _BPW)], om)
    pltpu.sync_copy(ac.at[pl.ds(s * _BPW, _BPW)], oc)
    pltpu.sync_copy(ow, aw_hbm.at[pl.ds(s * _BPW, _BPW)])
    pltpu.sync_copy(om, am_hbm.at[pl.ds(s * _BPW, _BPW)])
    pltpu.sync_copy(oc, ac_hbm.at[pl.ds(s * _BPW, _BPW)])


def _fin_body(aw_ref, am_ref, ac_ref, o_ref):
    aw = aw_ref[...]
    am = am_ref[...]
    ac = ac_ref[...]
    pres = ac > 0.0
    safe = jnp.where(pres, aw, 1.0)
    ratios = jnp.where(pres, am / safe, 0.0)
    count = jnp.sum(pres.astype(jnp.float32))
    o_ref[...] = jnp.full((1, 1), 100.0 * jnp.sum(ratios) / count)


@jax.jit
def kernel(beta, pred, particle_id, track_params, reconstructable):
    n = beta.shape[0]
    grain = _NSUB * _LANES
    npad = ((n + grain - 1) // grain) * grain
    rows = npad // grain
    chunk = npad // _NSUB
    padn = npad - n

    beta_p = jnp.pad(beta, (0, padn))
    pid_p = jnp.pad(particle_id.astype(jnp.int32), (0, padn))
    rec_p = jnp.pad(reconstructable.astype(jnp.int32), (0, padn))
    pred_t = jnp.pad(pred, ((0, padn), (0, 0))).T
    track_t = jnp.pad(track_params, ((0, padn), (0, 0))).T

    w, wm, c = pl.pallas_call(
        _tc_body,
        out_shape=[jax.ShapeDtypeStruct((npad,), jnp.float32)] * 3,
    )(beta_p, pred_t, track_t, pid_p, rec_p)

    mesh = plsc.VectorSubcoreMesh(
        core_axis_name="c", subcore_axis_name="s", num_cores=1
    )
    sc = pl.kernel(
        functools.partial(_sc_body, rows),
        out_type=[jax.ShapeDtypeStruct((_NUM_BINS,), jnp.float32)] * 3,
        mesh=mesh,
        scratch_types=[
            pltpu.VMEM((chunk,), jnp.int32),
            pltpu.VMEM((chunk,), jnp.float32),
            pltpu.VMEM((chunk,), jnp.float32),
            pltpu.VMEM((chunk,), jnp.float32),
            pltpu.VMEM((_BPW,), jnp.float32),
            pltpu.VMEM((_BPW,), jnp.float32),
            pltpu.VMEM((_BPW,), jnp.float32),
            pltpu.VMEM((_BPW,), jnp.float32),
            pltpu.VMEM_SHARED((_NUM_BINS,), jnp.float32),
            pltpu.VMEM_SHARED((_NUM_BINS,), jnp.float32),
            pltpu.VMEM_SHARED((_NUM_BINS,), jnp.float32),
        ],
    )
    aw, am, ac = sc(w, wm, c, pid_p)

    out = pl.pallas_call(
        _fin_body,
        out_shape=jax.ShapeDtypeStruct((1, 1), jnp.float32),
    )(aw, am, ac)
    return out[0, 0]
